# baseline (speedup 1.0000x reference)
import functools

import jax
import jax.numpy as jnp
from jax import lax
from jax.experimental import pallas as pl
from jax.experimental.pallas import tpu as pltpu
from jax.experimental.pallas import tpu_sc as plsc

N_MOL, E_MOL = 10000, 320000
N_PROT, E_PROT = 1000, 32000
B = 64
H, NH = 64, 4
HD = H // NH
NC, NS = 2, 16
NW = NC * NS
F32 = jnp.float32


def _make_agg(N, E, CH):
    EW = E // NW
    C = EW // CH
    assert EW % CH == 0 and (EW % 8) == 0
    NG = (CH + 7) // 8
    AP = 16 * NG
    GFULL = CH // 8
    ZCH = 200
    ZC = N // ZCH
    assert N % ZCH == 0
    ZITER = (ZC + NS - 1) // NS
    mesh = plsc.VectorSubcoreMesh(core_axis_name="c", subcore_axis_name="s")

    @functools.partial(
        pl.kernel,
        out_type=jax.ShapeDtypeStruct((NC, N, H), F32),
        mesh=mesh,
        scratch_types=[
            pltpu.VMEM((C, CH), jnp.int32),
            pltpu.VMEM((C, CH), jnp.int32),
            pltpu.VMEM((C, AP), F32),
            pltpu.VMEM((CH, H), F32),
            pltpu.VMEM((CH, H), F32),
            pltpu.VMEM((CH, H), F32),
            pltpu.VMEM((CH, H), F32),
            pltpu.VMEM((3, H), F32),
            pltpu.VMEM((ZCH, H), F32),
            pltpu.VMEM_SHARED((N, H), F32),
            pltpu.SemaphoreType.DMA,
            pltpu.SemaphoreType.DMA,
            pltpu.SemaphoreType.DMA,
            pltpu.SemaphoreType.DMA,
        ],
        compiler_params=pltpu.CompilerParams(use_tc_tiling_on_sc=False),
    )
    def k(x_hbm, src_hbm, dst_hbm, attr_hbm, wb_hbm, out_hbm,
          src_v, dst_v, attr_v, xg0, xg1, mg0, mg1, wb_v, stage_v, agg_sh,
          gsem0, gsem1, ssem0, ssem1):
        cid = lax.axis_index("c")
        sid = lax.axis_index("s")
        wid = cid * NS + sid

        pltpu.sync_copy(src_hbm.at[wid], src_v)
        pltpu.sync_copy(dst_hbm.at[wid], dst_v)
        pltpu.sync_copy(attr_hbm.at[wid], attr_v)
        pltpu.sync_copy(wb_hbm, wb_v)

        def zrow(r, carry):
            for i in range(4):
                stage_v[r, pl.ds(16 * i, 16)] = jnp.zeros((16,), F32)
            return carry
        lax.fori_loop(0, ZCH, zrow, 0)
        for kk in range(ZITER):
            zc = sid + kk * NS
            if ZC % NS == 0:
                pltpu.sync_copy(stage_v, agg_sh.at[pl.ds(zc * ZCH, ZCH)])
            else:
                @pl.when(zc < ZC)
                def _():
                    pltpu.sync_copy(stage_v, agg_sh.at[pl.ds(zc * ZCH, ZCH)])
        plsc.subcore_barrier()

        w0 = [wb_v[0, pl.ds(16 * i, 16)] for i in range(4)]
        w1 = [wb_v[1, pl.ds(16 * i, 16)] for i in range(4)]
        bb = [wb_v[2, pl.ds(16 * i, 16)] for i in range(4)]

        def gather(j, xg, gsem):
            pltpu.make_async_copy(x_hbm.at[src_v.at[j]], xg, gsem).start()

        def gather_wait(xg, gsem):
            pltpu.make_async_copy(x_hbm.at[src_v.at[0]], xg, gsem).wait()

        def scat(j, mg, ssem):
            pltpu.make_async_copy(
                mg, agg_sh.at[dst_v.at[j]], ssem).start(add=True)

        def scat_wait(mg, ssem):
            pltpu.make_async_copy(
                mg, agg_sh.at[dst_v.at[0]], ssem).wait()

        def compute(j, xg, mg):
            def do_edges(base, va, nvalid):
                for i in range(nvalid):
                    e = base + i
                    a0 = va[2 * i]
                    a1 = va[2 * i + 1]
                    for t in range(4):
                        v = xg[e, pl.ds(16 * t, 16)]
                        mg[e, pl.ds(16 * t, 16)] = jnp.maximum(
                            v + a0 * w0[t] + a1 * w1[t] + bb[t], 0.0)

            def grp_body(g, c2):
                do_edges(g * 8, attr_v[j, pl.ds(16 * g, 16)], 8)
                return c2
            lax.fori_loop(0, GFULL, grp_body, 0, unroll=2)
            if CH % 8:
                do_edges(GFULL * 8, attr_v[j, pl.ds(16 * GFULL, 16)], CH % 8)

        G2 = C // 2
        assert C % 2 == 0
        gather(0, xg0, gsem0)

        def pipe_body(g, carry):
            j0 = 2 * g
            j1 = 2 * g + 1
            gather(j1, xg1, gsem1)
            gather_wait(xg0, gsem0)
            compute(j0, xg0, mg0)
            pltpu.sync_copy(mg0, agg_sh.at[dst_v.at[j0]], add=True)

            @pl.when(g < G2 - 1)
            def _():
                gather(j0 + 2, xg0, gsem0)
            gather_wait(xg1, gsem1)
            compute(j1, xg1, mg1)
            pltpu.sync_copy(mg1, agg_sh.at[dst_v.at[j1]], add=True)
            return carry
        lax.fori_loop(0, G2, pipe_body, 0)
        plsc.subcore_barrier()

        for kk in range(ZITER):
            zc = sid + kk * NS
            if ZC % NS == 0:
                pltpu.sync_copy(agg_sh.at[pl.ds(zc * ZCH, ZCH)], stage_v)
                pltpu.sync_copy(stage_v, out_hbm.at[cid, pl.ds(zc * ZCH, ZCH)])
            else:
                @pl.when(zc < ZC)
                def _():
                    pltpu.sync_copy(agg_sh.at[pl.ds(zc * ZCH, ZCH)], stage_v)
                    pltpu.sync_copy(stage_v, out_hbm.at[cid, pl.ds(zc * ZCH, ZCH)])

    return k


_agg_mol = _make_agg(N_MOL, E_MOL, 100)
_agg_prot = _make_agg(N_PROT, E_PROT, 100)


def _dot(a, b):
    return jax.lax.dot_general(a, b, (((1,), (0,)), ((), ())),
                               preferred_element_type=F32)


def _prelude_body(mx_ref, mw_ref, mb_ref, px_ref, pw_ref, pb_ref,
                  om_ref, op_ref):
    om_ref[...] = _dot(mx_ref[...], mw_ref[...]) + mb_ref[...]
    op_ref[...] = _dot(px_ref[...], pw_ref[...]) + pb_ref[...]


def _prelude(mx, mw, mb, px, pw, pb):
    return pl.pallas_call(
        _prelude_body,
        out_shape=[jax.ShapeDtypeStruct((N_MOL, H), F32),
                   jax.ShapeDtypeStruct((N_PROT, H), F32)],
    )(mx, mw, mb, px, pw, pb)


def _gine_mlp_body(x_ref, agg_ref, w1_ref, b1_ref, w2_ref, b2_ref, o_ref):
    h = x_ref[...] + agg_ref[0] + agg_ref[1]
    h = jnp.maximum(_dot(h, w1_ref[...]) + b1_ref[...], 0.0)
    o_ref[...] = jnp.maximum(_dot(h, w2_ref[...]) + b2_ref[...], 0.0)


def _gine_mlp(x, agg, w1, b1, w2, b2):
    n = x.shape[0]
    return pl.pallas_call(
        _gine_mlp_body,
        out_shape=jax.ShapeDtypeStruct((n, H), F32),
    )(x, agg, w1, b1, w2, b2)


def _qkv_body(hm_ref, hp_ref,
              wqm_ref, bqm_ref, wkp_ref, bkp_ref, wvp_ref, bvp_ref,
              wqp_ref, bqp_ref, wkm_ref, bkm_ref, wvm_ref, bvm_ref,
              qm_ref, kp_ref, vp_ref, qp_ref, km_ref, vm_ref):
    hm = hm_ref[...]
    hp = hp_ref[...]
    qm_ref[...] = _dot(hm, wqm_ref[...]) + bqm_ref[...]
    kp_ref[...] = _dot(hp, wkp_ref[...]) + bkp_ref[...]
    vp_ref[...] = _dot(hp, wvp_ref[...]) + bvp_ref[...]
    qp_ref[...] = _dot(hp, wqp_ref[...]) + bqp_ref[...]
    km_ref[...] = _dot(hm, wkm_ref[...]) + bkm_ref[...]
    vm_ref[...] = _dot(hm, wvm_ref[...]) + bvm_ref[...]


def _qkv(hm, hp_pad, wqm, bqm, wkp, bkp, wvp, bvp, wqp, bqp, wkm, bkm,
         wvm, bvm):
    np_pad = hp_pad.shape[0]
    return pl.pallas_call(
        _qkv_body,
        out_shape=[jax.ShapeDtypeStruct((N_MOL, H), F32),
                   jax.ShapeDtypeStruct((np_pad, H), F32),
                   jax.ShapeDtypeStruct((np_pad, H), F32),
                   jax.ShapeDtypeStruct((np_pad, H), F32),
                   jax.ShapeDtypeStruct((N_MOL, H), F32),
                   jax.ShapeDtypeStruct((N_MOL, H), F32)],
    )(hm, hp_pad, wqm, bqm, wkp, bkp, wvp, bvp, wqp, bqp, wkm, bkm, wvm, bvm)


def _attn_body(nk_real, q_ref, k_ref, v_ref, res_ref, o_ref):
    q = q_ref[...]
    k = k_ref[...]
    v = v_ref[...]
    nk = k.shape[0]
    scale = 1.0 / (HD ** 0.5)
    need_mask = nk_real < nk
    if need_mask:
        kmask = lax.broadcasted_iota(jnp.int32, (1, nk), 1) < nk_real
    outs = []
    for h in range(NH):
        qh = q[:, h * HD:(h + 1) * HD] * scale
        kh = k[:, h * HD:(h + 1) * HD]
        s = jax.lax.dot_general(qh, kh, (((1,), (1,)), ((), ())),
                                preferred_element_type=F32)
        if need_mask:
            s = jnp.where(kmask, s, -1e30)
        m = jnp.max(s, axis=1, keepdims=True)
        e = jnp.exp(s - m)
        w = e / jnp.sum(e, axis=1, keepdims=True)
        outs.append(_dot(w, v[:, h * HD:(h + 1) * HD]))
    o_ref[...] = res_ref[...] + jnp.concatenate(outs, axis=1)


def _attn(q, kk, vv, res, bq, nk_real):
    nq = q.shape[0]
    nk = kk.shape[0]
    grid = (nq // bq,)
    qspec = pl.BlockSpec((bq, H), lambda i: (i, 0))
    kspec = pl.BlockSpec((nk, H), lambda i: (0, 0))
    return pl.pallas_call(
        functools.partial(_attn_body, nk_real),
        grid=grid,
        in_specs=[qspec, kspec, kspec, qspec],
        out_specs=qspec,
        out_shape=jax.ShapeDtypeStruct((nq, H), F32),
        compiler_params=pltpu.CompilerParams(
            dimension_semantics=("arbitrary",)),
    )(q, kk, vv, res)


def _pool_head_body(hm_ref, hp_ref, mb_ref, pb_ref,
                    w1_ref, b1_ref, w2_ref, b2_ref, o_ref):
    def seg_mean(h, batch, n):
        iota = lax.broadcasted_iota(jnp.int32, (n, B), 1)
        oh = (batch == iota).astype(F32)
        s = jax.lax.dot_general(oh, h, (((0,), (0,)), ((), ())),
                                preferred_element_type=F32)
        ones = jnp.ones((n, 1), F32)
        cnt = jax.lax.dot_general(oh, ones, (((0,), (0,)), ((), ())),
                                  preferred_element_type=F32)
        return s / jnp.maximum(cnt, 1.0)
    zm = seg_mean(hm_ref[...], mb_ref[...], N_MOL)
    zp = seg_mean(hp_ref[...], pb_ref[...], N_PROT)
    z = jnp.concatenate([zm, zp], axis=1)
    x = jnp.maximum(_dot(z, w1_ref[...]) + b1_ref[...], 0.0)
    y = _dot(x, w2_ref[...]) + b2_ref[...]
    o_ref[...] = 1.0 / (1.0 + jnp.exp(-y))


def _pool_head(hm, hp, mbatch, pbatch, w1, b1, w2, b2):
    return pl.pallas_call(
        _pool_head_body,
        out_shape=jax.ShapeDtypeStruct((B, 1), F32),
    )(hm, hp, mbatch, pbatch, w1, b1, w2, b2)


def kernel(mol_x, mol_edge_index, mol_edge_attr, mol_batch, prot_x,
           prot_edge_index, prot_edge_attr, prot_batch, mol_node_W,
           mol_node_b, prot_node_W, prot_node_b, mol_edge_W, mol_edge_b,
           prot_edge_W, prot_edge_b, mol_c1_W1, mol_c1_b1, mol_c1_W2,
           mol_c1_b2, mol_c2_W1, mol_c2_b1, mol_c2_W2, mol_c2_b2,
           prot_c1_W1, prot_c1_b1, prot_c1_W2, prot_c1_b2, prot_c2_W1,
           prot_c2_b1, prot_c2_W2, prot_c2_b2, mp_WQ, mp_bQ, mp_WK, mp_bK,
           mp_WV, mp_bV, pm_WQ, pm_bQ, pm_WK, pm_bK, pm_WV, pm_bV,
           fc1_W, fc1_b, fc2_W, fc2_b):
    r1 = lambda b: b.reshape(1, -1)

    ch = 100
    ap = 16 * ((ch + 7) // 8)

    def attr_layout(a):
        a = a.reshape(NW, -1, 2 * ch)
        return jnp.pad(a, ((0, 0), (0, 0), (0, ap - 2 * ch)))
    ms = mol_edge_index[0].reshape(NW, -1, ch)
    md = mol_edge_index[1].reshape(NW, -1, ch)
    ma = attr_layout(mol_edge_attr)
    ps = prot_edge_index[0].reshape(NW, -1, ch)
    pd = prot_edge_index[1].reshape(NW, -1, ch)
    pa = attr_layout(prot_edge_attr)
    wb_m = jnp.concatenate([mol_edge_W, r1(mol_edge_b)], axis=0)
    wb_p = jnp.concatenate([prot_edge_W, r1(prot_edge_b)], axis=0)

    x0m, x0p = _prelude(mol_x, mol_node_W, r1(mol_node_b),
                        prot_x, prot_node_W, r1(prot_node_b))

    h = x0m
    for w1, b1, w2, b2 in ((mol_c1_W1, mol_c1_b1, mol_c1_W2, mol_c1_b2),
                           (mol_c2_W1, mol_c2_b1, mol_c2_W2, mol_c2_b2)):
        agg = _agg_mol(h, ms, md, ma, wb_m)
        h = _gine_mlp(h, agg, w1, r1(b1), w2, r1(b2))
    hm = h

    h = x0p
    for w1, b1, w2, b2 in ((prot_c1_W1, prot_c1_b1, prot_c1_W2, prot_c1_b2),
                           (prot_c2_W1, prot_c2_b1, prot_c2_W2, prot_c2_b2)):
        agg = _agg_prot(h, ps, pd, pa, wb_p)
        h = _gine_mlp(h, agg, w1, r1(b1), w2, r1(b2))
    hp = h

    hp_pad = jnp.pad(hp, ((0, 1024 - N_PROT), (0, 0)))
    qm, kp, vp, qp, km, vm = _qkv(
        hm, hp_pad, mp_WQ, r1(mp_bQ), mp_WK, r1(mp_bK), mp_WV, r1(mp_bV),
        pm_WQ, r1(pm_bQ), pm_WK, r1(pm_bK), pm_WV, r1(pm_bV))

    hm2 = _attn(qm, kp, vp, hm, 1000, N_PROT)
    hp2_pad = _attn(qp, km, vm, hp_pad, 128, N_MOL)
    hp2 = hp2_pad[:N_PROT]

    out = _pool_head(hm2, hp2, mol_batch.reshape(-1, 1),
                     prot_batch.reshape(-1, 1),
                     fc1_W, r1(fc1_b), fc2_W, r1(fc2_b))
    return out.reshape(B)


# speedup vs baseline: 1.3446x; 1.3446x over previous
import functools

import jax
import jax.numpy as jnp
from jax import lax
from jax.experimental import pallas as pl
from jax.experimental.pallas import tpu as pltpu
from jax.experimental.pallas import tpu_sc as plsc

N_MOL, E_MOL = 10000, 320000
N_PROT, E_PROT = 1000, 32000
B = 64
H, NH = 64, 4
HD = H // NH
NC, NS = 2, 16
NW = NC * NS
F32 = jnp.float32


def _make_agg(N, E, CH):
    EW = E // NW
    C = EW // CH
    assert EW % CH == 0 and (EW % 8) == 0
    NG = (CH + 7) // 8
    AP = 16 * NG
    GFULL = CH // 8
    ZCH = 200
    ZC = N // ZCH
    assert N % ZCH == 0
    ZITER = (ZC + NS - 1) // NS
    mesh = plsc.VectorSubcoreMesh(core_axis_name="c", subcore_axis_name="s")

    @functools.partial(
        pl.kernel,
        out_type=jax.ShapeDtypeStruct((NC, N, H), F32),
        mesh=mesh,
        scratch_types=[
            pltpu.VMEM((C, CH), jnp.int32),
            pltpu.VMEM((C, CH), jnp.int32),
            pltpu.VMEM((C, AP), F32),
            pltpu.VMEM((CH, H), F32),
            pltpu.VMEM((CH, H), F32),
            pltpu.VMEM((CH, H), F32),
            pltpu.VMEM((CH, H), F32),
            pltpu.VMEM((3, H), F32),
            pltpu.VMEM((ZCH, H), F32),
            pltpu.VMEM_SHARED((N, H), F32),
            pltpu.SemaphoreType.DMA,
            pltpu.SemaphoreType.DMA,
            pltpu.SemaphoreType.DMA,
            pltpu.SemaphoreType.DMA,
        ],
        compiler_params=pltpu.CompilerParams(use_tc_tiling_on_sc=False),
    )
    def k(x_hbm, src_hbm, dst_hbm, attr_hbm, wb_hbm, out_hbm,
          src_v, dst_v, attr_v, xg0, xg1, mg0, mg1, wb_v, stage_v, agg_sh,
          gsem0, gsem1, ssem0, ssem1):
        cid = lax.axis_index("c")
        sid = lax.axis_index("s")
        wid = cid * NS + sid

        pltpu.sync_copy(src_hbm.at[wid], src_v)
        pltpu.sync_copy(dst_hbm.at[wid], dst_v)
        pltpu.sync_copy(attr_hbm.at[wid], attr_v)
        pltpu.sync_copy(wb_hbm, wb_v)

        def zrow(r, carry):
            for i in range(4):
                stage_v[r, pl.ds(16 * i, 16)] = jnp.zeros((16,), F32)
            return carry
        lax.fori_loop(0, ZCH, zrow, 0)
        for kk in range(ZITER):
            zc = sid + kk * NS
            if ZC % NS == 0:
                pltpu.sync_copy(stage_v, agg_sh.at[pl.ds(zc * ZCH, ZCH)])
            else:
                @pl.when(zc < ZC)
                def _():
                    pltpu.sync_copy(stage_v, agg_sh.at[pl.ds(zc * ZCH, ZCH)])
        plsc.subcore_barrier()

        w0 = [wb_v[0, pl.ds(16 * i, 16)] for i in range(4)]
        w1 = [wb_v[1, pl.ds(16 * i, 16)] for i in range(4)]
        bb = [wb_v[2, pl.ds(16 * i, 16)] for i in range(4)]

        def gather(j, xg, gsem):
            pltpu.make_async_copy(x_hbm.at[src_v.at[j]], xg, gsem).start()

        def gather_wait(xg, gsem):
            pltpu.make_async_copy(x_hbm.at[src_v.at[0]], xg, gsem).wait()

        def scat(j, mg, ssem):
            pltpu.make_async_copy(
                mg, agg_sh.at[dst_v.at[j]], ssem).start(add=True)

        def scat_wait(mg, ssem):
            pltpu.make_async_copy(
                mg, agg_sh.at[dst_v.at[0]], ssem).wait()

        def compute(j, xg, mg):
            def do_edges(base, va, nvalid):
                for i in range(nvalid):
                    e = base + i
                    a0 = va[2 * i]
                    a1 = va[2 * i + 1]
                    for t in range(4):
                        v = xg[e, pl.ds(16 * t, 16)]
                        mg[e, pl.ds(16 * t, 16)] = jnp.maximum(
                            v + a0 * w0[t] + a1 * w1[t] + bb[t], 0.0)

            def grp_body(g, c2):
                do_edges(g * 8, attr_v[j, pl.ds(16 * g, 16)], 8)
                return c2
            lax.fori_loop(0, GFULL, grp_body, 0)
            if CH % 8:
                do_edges(GFULL * 8, attr_v[j, pl.ds(16 * GFULL, 16)], CH % 8)

        def chunk_body(j, carry):
            gather(j, xg0, gsem0)
            gather_wait(xg0, gsem0)
            compute(j, xg0, mg0)
            pltpu.sync_copy(mg0, agg_sh.at[dst_v.at[j]], add=True)
            return carry
        lax.fori_loop(0, C, chunk_body, 0)
        plsc.subcore_barrier()

        for kk in range(ZITER):
            zc = sid + kk * NS
            if ZC % NS == 0:
                pltpu.sync_copy(agg_sh.at[pl.ds(zc * ZCH, ZCH)], stage_v)
                pltpu.sync_copy(stage_v, out_hbm.at[cid, pl.ds(zc * ZCH, ZCH)])
            else:
                @pl.when(zc < ZC)
                def _():
                    pltpu.sync_copy(agg_sh.at[pl.ds(zc * ZCH, ZCH)], stage_v)
                    pltpu.sync_copy(stage_v, out_hbm.at[cid, pl.ds(zc * ZCH, ZCH)])

    return k


_agg_mol = _make_agg(N_MOL, E_MOL, 100)
_agg_prot = _make_agg(N_PROT, E_PROT, 100)


def _dot(a, b):
    return jax.lax.dot_general(a, b, (((1,), (0,)), ((), ())),
                               preferred_element_type=F32)


def _prelude_body(mx_ref, mw_ref, mb_ref, px_ref, pw_ref, pb_ref,
                  om_ref, op_ref):
    om_ref[...] = _dot(mx_ref[...], mw_ref[...]) + mb_ref[...]
    op_ref[...] = _dot(px_ref[...], pw_ref[...]) + pb_ref[...]


def _prelude(mx, mw, mb, px, pw, pb):
    return pl.pallas_call(
        _prelude_body,
        out_shape=[jax.ShapeDtypeStruct((N_MOL, H), F32),
                   jax.ShapeDtypeStruct((N_PROT, H), F32)],
    )(mx, mw, mb, px, pw, pb)


def _gine_mlp_body(x_ref, agg_ref, w1_ref, b1_ref, w2_ref, b2_ref, o_ref):
    h = x_ref[...] + agg_ref[0] + agg_ref[1]
    h = jnp.maximum(_dot(h, w1_ref[...]) + b1_ref[...], 0.0)
    o_ref[...] = jnp.maximum(_dot(h, w2_ref[...]) + b2_ref[...], 0.0)


def _gine_mlp(x, agg, w1, b1, w2, b2):
    n = x.shape[0]
    return pl.pallas_call(
        _gine_mlp_body,
        out_shape=jax.ShapeDtypeStruct((n, H), F32),
    )(x, agg, w1, b1, w2, b2)


def _qkv_body(hm_ref, hp_ref,
              wqm_ref, bqm_ref, wkp_ref, bkp_ref, wvp_ref, bvp_ref,
              wqp_ref, bqp_ref, wkm_ref, bkm_ref, wvm_ref, bvm_ref,
              qm_ref, kp_ref, vp_ref, qp_ref, km_ref, vm_ref):
    hm = hm_ref[...]
    hp = hp_ref[...]
    qm_ref[...] = _dot(hm, wqm_ref[...]) + bqm_ref[...]
    kp_ref[...] = _dot(hp, wkp_ref[...]) + bkp_ref[...]
    vp_ref[...] = _dot(hp, wvp_ref[...]) + bvp_ref[...]
    qp_ref[...] = _dot(hp, wqp_ref[...]) + bqp_ref[...]
    km_ref[...] = _dot(hm, wkm_ref[...]) + bkm_ref[...]
    vm_ref[...] = _dot(hm, wvm_ref[...]) + bvm_ref[...]


def _qkv(hm, hp_pad, wqm, bqm, wkp, bkp, wvp, bvp, wqp, bqp, wkm, bkm,
         wvm, bvm):
    np_pad = hp_pad.shape[0]
    return pl.pallas_call(
        _qkv_body,
        out_shape=[jax.ShapeDtypeStruct((N_MOL, H), F32),
                   jax.ShapeDtypeStruct((np_pad, H), F32),
                   jax.ShapeDtypeStruct((np_pad, H), F32),
                   jax.ShapeDtypeStruct((np_pad, H), F32),
                   jax.ShapeDtypeStruct((N_MOL, H), F32),
                   jax.ShapeDtypeStruct((N_MOL, H), F32)],
    )(hm, hp_pad, wqm, bqm, wkp, bkp, wvp, bvp, wqp, bqp, wkm, bkm, wvm, bvm)


def _attn_body(nk_real, q_ref, k_ref, v_ref, res_ref, o_ref):
    q = q_ref[...]
    k = k_ref[...]
    v = v_ref[...]
    nk = k.shape[0]
    scale = 1.0 / (HD ** 0.5)
    need_mask = nk_real < nk
    if need_mask:
        kmask = lax.broadcasted_iota(jnp.int32, (1, nk), 1) < nk_real
    outs = []
    for h in range(NH):
        qh = q[:, h * HD:(h + 1) * HD] * scale
        kh = k[:, h * HD:(h + 1) * HD]
        s = jax.lax.dot_general(qh, kh, (((1,), (1,)), ((), ())),
                                preferred_element_type=F32)
        if need_mask:
            s = jnp.where(kmask, s, -1e30)
        m = jnp.max(s, axis=1, keepdims=True)
        e = jnp.exp(s - m)
        w = e / jnp.sum(e, axis=1, keepdims=True)
        outs.append(_dot(w, v[:, h * HD:(h + 1) * HD]))
    o_ref[...] = res_ref[...] + jnp.concatenate(outs, axis=1)


def _attn(q, kk, vv, res, bq, nk_real):
    nq = q.shape[0]
    nk = kk.shape[0]
    grid = (nq // bq,)
    qspec = pl.BlockSpec((bq, H), lambda i: (i, 0))
    kspec = pl.BlockSpec((nk, H), lambda i: (0, 0))
    return pl.pallas_call(
        functools.partial(_attn_body, nk_real),
        grid=grid,
        in_specs=[qspec, kspec, kspec, qspec],
        out_specs=qspec,
        out_shape=jax.ShapeDtypeStruct((nq, H), F32),
        compiler_params=pltpu.CompilerParams(
            dimension_semantics=("arbitrary",)),
    )(q, kk, vv, res)


def _pool_head_body(hm_ref, hp_ref, mb_ref, pb_ref,
                    w1_ref, b1_ref, w2_ref, b2_ref, o_ref):
    def seg_mean(h, batch, n):
        iota = lax.broadcasted_iota(jnp.int32, (n, B), 1)
        oh = (batch == iota).astype(F32)
        s = jax.lax.dot_general(oh, h, (((0,), (0,)), ((), ())),
                                preferred_element_type=F32)
        ones = jnp.ones((n, 1), F32)
        cnt = jax.lax.dot_general(oh, ones, (((0,), (0,)), ((), ())),
                                  preferred_element_type=F32)
        return s / jnp.maximum(cnt, 1.0)
    zm = seg_mean(hm_ref[...], mb_ref[...], N_MOL)
    zp = seg_mean(hp_ref[...], pb_ref[...], N_PROT)
    z = jnp.concatenate([zm, zp], axis=1)
    x = jnp.maximum(_dot(z, w1_ref[...]) + b1_ref[...], 0.0)
    y = _dot(x, w2_ref[...]) + b2_ref[...]
    o_ref[...] = 1.0 / (1.0 + jnp.exp(-y))


def _pool_head(hm, hp, mbatch, pbatch, w1, b1, w2, b2):
    return pl.pallas_call(
        _pool_head_body,
        out_shape=jax.ShapeDtypeStruct((B, 1), F32),
    )(hm, hp, mbatch, pbatch, w1, b1, w2, b2)


def kernel(mol_x, mol_edge_index, mol_edge_attr, mol_batch, prot_x,
           prot_edge_index, prot_edge_attr, prot_batch, mol_node_W,
           mol_node_b, prot_node_W, prot_node_b, mol_edge_W, mol_edge_b,
           prot_edge_W, prot_edge_b, mol_c1_W1, mol_c1_b1, mol_c1_W2,
           mol_c1_b2, mol_c2_W1, mol_c2_b1, mol_c2_W2, mol_c2_b2,
           prot_c1_W1, prot_c1_b1, prot_c1_W2, prot_c1_b2, prot_c2_W1,
           prot_c2_b1, prot_c2_W2, prot_c2_b2, mp_WQ, mp_bQ, mp_WK, mp_bK,
           mp_WV, mp_bV, pm_WQ, pm_bQ, pm_WK, pm_bK, pm_WV, pm_bV,
           fc1_W, fc1_b, fc2_W, fc2_b):
    r1 = lambda b: b.reshape(1, -1)

    ch = 100
    ap = 16 * ((ch + 7) // 8)

    def attr_layout(a):
        a = a.reshape(NW, -1, 2 * ch)
        return jnp.pad(a, ((0, 0), (0, 0), (0, ap - 2 * ch)))
    ms = mol_edge_index[0].reshape(NW, -1, ch)
    md = mol_edge_index[1].reshape(NW, -1, ch)
    ma = attr_layout(mol_edge_attr)
    ps = prot_edge_index[0].reshape(NW, -1, ch)
    pd = prot_edge_index[1].reshape(NW, -1, ch)
    pa = attr_layout(prot_edge_attr)
    wb_m = jnp.concatenate([mol_edge_W, r1(mol_edge_b)], axis=0)
    wb_p = jnp.concatenate([prot_edge_W, r1(prot_edge_b)], axis=0)

    x0m, x0p = _prelude(mol_x, mol_node_W, r1(mol_node_b),
                        prot_x, prot_node_W, r1(prot_node_b))

    h = x0m
    for w1, b1, w2, b2 in ((mol_c1_W1, mol_c1_b1, mol_c1_W2, mol_c1_b2),
                           (mol_c2_W1, mol_c2_b1, mol_c2_W2, mol_c2_b2)):
        agg = _agg_mol(h, ms, md, ma, wb_m)
        h = _gine_mlp(h, agg, w1, r1(b1), w2, r1(b2))
    hm = h

    h = x0p
    for w1, b1, w2, b2 in ((prot_c1_W1, prot_c1_b1, prot_c1_W2, prot_c1_b2),
                           (prot_c2_W1, prot_c2_b1, prot_c2_W2, prot_c2_b2)):
        agg = _agg_prot(h, ps, pd, pa, wb_p)
        h = _gine_mlp(h, agg, w1, r1(b1), w2, r1(b2))
    hp = h

    hp_pad = jnp.pad(hp, ((0, 1024 - N_PROT), (0, 0)))
    qm, kp, vp, qp, km, vm = _qkv(
        hm, hp_pad, mp_WQ, r1(mp_bQ), mp_WK, r1(mp_bK), mp_WV, r1(mp_bV),
        pm_WQ, r1(pm_bQ), pm_WK, r1(pm_bK), pm_WV, r1(pm_bV))

    hm2 = _attn(qm, kp, vp, hm, 1000, N_PROT)
    hp2_pad = _attn(qp, km, vm, hp_pad, 128, N_MOL)
    hp2 = hp2_pad[:N_PROT]

    out = _pool_head(hm2, hp2, mol_batch.reshape(-1, 1),
                     prot_batch.reshape(-1, 1),
                     fc1_W, r1(fc1_b), fc2_W, r1(fc2_b))
    return out.reshape(B)
